# baseline (speedup 1.0000x reference)
import jax
import jax.numpy as jnp
from jax import lax
from jax.experimental import pallas as pl
from jax.experimental.pallas import tpu as pltpu
from jax.experimental.pallas import tpu_sc as plsc

XN = 4096 * 200
D = 10
VOC = 100
NC, NS = 2, 16
NW = NC * NS
CH = 128
NCH = 25
BI = CH * NCH
CHUNK = XN // NW
NB = CHUNK // BI
NROWS = XN // CH
RPB = BI // CH


def _sc_body(x_hbm, tbl_hbm, out_hbm, x_v, rows_v, sem):
    wid = lax.axis_index("s") * NC + lax.axis_index("c")
    row_base = wid * (CHUNK // CH)

    def do_block(b, _):
        row0 = row_base + b * RPB
        pltpu.sync_copy(x_hbm.at[pl.ds(row0, RPB), :], x_v)
        copies = []
        for c in range(NCH):
            copies.append(pltpu.async_copy(
                tbl_hbm.at[x_v.at[c]],
                rows_v.at[c],
                sem,
            ))
        for cp in copies:
            cp.wait()
        pltpu.sync_copy(rows_v, out_hbm.at[pl.ds(row0, RPB), :, :])
        return 0

    lax.fori_loop(0, NB, do_block, 0)


@jax.jit
def _sc_lookup(x2d, tbl):
    mesh = plsc.VectorSubcoreMesh(core_axis_name="c", subcore_axis_name="s")
    f = pl.kernel(
        _sc_body,
        mesh=mesh,
        out_type=jax.ShapeDtypeStruct((NROWS, CH, D), jnp.float32),
        scratch_types=[
            pltpu.VMEM((RPB, CH), jnp.int32),
            pltpu.VMEM((RPB, CH, D), jnp.float32),
            pltpu.SemaphoreType.DMA,
        ],
        compiler_params=pltpu.CompilerParams(
            needs_layout_passes=False, use_tc_tiling_on_sc=False),
    )
    return f(x2d, tbl)


def kernel(x, table):
    out = _sc_lookup(x.reshape(NROWS, CH), table)
    return out.reshape(x.shape[0], x.shape[1], D)


# speedup vs baseline: 1.4475x; 1.4475x over previous
import jax
import jax.numpy as jnp
from jax import lax
from jax.experimental import pallas as pl
from jax.experimental.pallas import tpu as pltpu
from jax.experimental.pallas import tpu_sc as plsc

XN = 4096 * 200
D = 10
VOC = 100
NC, NS, L = 2, 16, 16
NW = NC * NS
CHUNK = XN // NW
BI = 3200
NB = CHUNK // BI


def _sc_body(x_hbm, tbl_hbm, out_hbm, x_v, tbl_v, out_v):
    wid = lax.axis_index("s") * NC + lax.axis_index("c")
    base = wid * CHUNK

    pltpu.sync_copy(tbl_hbm, tbl_v)

    iota_d = lax.iota(jnp.int32, L) * D

    def do_block(b, _):
        off = base + b * BI
        pltpu.sync_copy(x_hbm.at[pl.ds(off, BI)], x_v)

        @plsc.parallel_loop(0, BI, step=L, unroll=8)
        def _group(o):
            idx16 = x_v[pl.ds(o, L)]
            tb = idx16 * D
            ob = iota_d + o * D
            for d in range(D):
                v = plsc.load_gather(tbl_v, [tb + d])
                plsc.store_scatter(out_v, [ob + d], v)

        pltpu.sync_copy(out_v, out_hbm.at[pl.ds(off * D, BI * D)])
        return 0

    lax.fori_loop(0, NB, do_block, 0)


@jax.jit
def _sc_lookup(x_flat, tbl_flat):
    mesh = plsc.VectorSubcoreMesh(core_axis_name="c", subcore_axis_name="s")
    f = pl.kernel(
        _sc_body,
        mesh=mesh,
        out_type=jax.ShapeDtypeStruct((XN * D,), jnp.float32),
        scratch_types=[
            pltpu.VMEM((BI,), jnp.int32),
            pltpu.VMEM((VOC * D,), jnp.float32),
            pltpu.VMEM((BI * D,), jnp.float32),
        ],
        compiler_params=pltpu.CompilerParams(needs_layout_passes=False),
    )
    return f(x_flat, tbl_flat)


def kernel(x, table):
    out_flat = _sc_lookup(x.reshape(-1), table.reshape(-1))
    return out_flat.reshape(x.shape[0], x.shape[1], D)


# speedup vs baseline: 14.8675x; 10.2713x over previous
import jax
import jax.numpy as jnp
from jax import lax
from jax.experimental import pallas as pl
from jax.experimental.pallas import tpu as pltpu
from jax.experimental.pallas import tpu_sc as plsc

NI = 4096
NJ = 200
D = 10
VOC = 100
NC, NS, L = 2, 16, 16
NW = NC * NS
IB = NI // NW
JB = 40
NB = NJ // JB
M = IB // L


def _sc_body(x_hbm, tbl_hbm, out_hbm, x_v, tbl_v, out_v, sem_x, sem_o):
    wid = lax.axis_index("s") * NC + lax.axis_index("c")
    i0 = wid * IB

    pltpu.sync_copy(tbl_hbm, tbl_v)

    def start_x(b, p):
        return pltpu.async_copy(
            x_hbm.at[pl.ds(b * JB, JB), pl.ds(i0, IB)], x_v.at[p], sem_x)

    x_copies = [start_x(0, 0)]
    out_copies = []

    for b in range(NB):
        p = b % 2
        if b + 1 < NB:
            x_copies.append(start_x(b + 1, (b + 1) % 2))
        x_copies[b].wait()
        if b >= 2:
            for cp in out_copies[(b - 2) * D:(b - 1) * D]:
                cp.wait()

        @plsc.parallel_loop(0, JB, unroll=2)
        def _row(j):
            for m in range(M):
                idx16 = x_v[p, j, pl.ds(m * L, L)]
                tb = idx16 * D
                for d in range(D):
                    v = plsc.load_gather(tbl_v, [tb + d])
                    out_v[p, d, j, pl.ds(m * L, L)] = v

        for d in range(D):
            out_copies.append(pltpu.async_copy(
                out_v.at[p, d],
                out_hbm.at[d, pl.ds(b * JB, JB), pl.ds(i0, IB)],
                sem_o))

    for cp in out_copies[(NB - 2) * D:]:
        cp.wait()


@jax.jit
def _sc_lookup(x_t, tbl_flat):
    mesh = plsc.VectorSubcoreMesh(core_axis_name="c", subcore_axis_name="s")
    f = pl.kernel(
        _sc_body,
        mesh=mesh,
        out_type=jax.ShapeDtypeStruct((D, NJ, NI), jnp.float32),
        scratch_types=[
            pltpu.VMEM((2, JB, IB), jnp.int32),
            pltpu.VMEM((VOC * D,), jnp.float32),
            pltpu.VMEM((2, D, JB, IB), jnp.float32),
            pltpu.SemaphoreType.DMA,
            pltpu.SemaphoreType.DMA,
        ],
        compiler_params=pltpu.CompilerParams(needs_layout_passes=False),
    )
    return f(x_t, tbl_flat)


def kernel(x, table):
    out_t = _sc_lookup(x.T, table.reshape(-1))
    return out_t.transpose(2, 1, 0)


# speedup vs baseline: 20.5711x; 1.3836x over previous
import jax
import jax.numpy as jnp
from jax import lax
from jax.experimental import pallas as pl
from jax.experimental.pallas import tpu as pltpu
from jax.experimental.pallas import tpu_sc as plsc

NI = 4096
NJ = 200
D = 10
VOC = 100
VP = 104
NC, NS, L = 2, 16, 16
NW = NC * NS
IB = NI // NW
JB = 40
NB = NJ // JB
M = IB // L


def _sc_body(x_hbm, tbl_hbm, out_hbm, x_v, tbl_v, out_v, sem_x, sem_o):
    wid = lax.axis_index("s") * NC + lax.axis_index("c")
    i0 = wid * IB

    pltpu.sync_copy(tbl_hbm, tbl_v)

    def start_x(b, p):
        return pltpu.async_copy(
            x_hbm.at[pl.ds(b * JB, JB), pl.ds(i0, IB)], x_v.at[p], sem_x)

    x_copies = [start_x(0, 0)]
    out_copies = []

    for b in range(NB):
        p = b % 2
        if b + 1 < NB:
            x_copies.append(start_x(b + 1, (b + 1) % 2))
        x_copies[b].wait()
        if b >= 2:
            for cp in out_copies[(b - 2) * D:(b - 1) * D]:
                cp.wait()

        @plsc.parallel_loop(0, JB, unroll=2)
        def _row(j):
            for m in range(M):
                idx16 = x_v[p, j, pl.ds(m * L, L)]
                for d in range(D):
                    v = plsc.load_gather(tbl_v.at[d], [idx16])
                    out_v[p, d, j, pl.ds(m * L, L)] = v

        for d in range(D):
            out_copies.append(pltpu.async_copy(
                out_v.at[p, d],
                out_hbm.at[d, pl.ds(b * JB, JB), pl.ds(i0, IB)],
                sem_o))

    for cp in out_copies[(NB - 2) * D:]:
        cp.wait()


@jax.jit
def _sc_lookup(x_t, tbl_t):
    mesh = plsc.VectorSubcoreMesh(core_axis_name="c", subcore_axis_name="s")
    f = pl.kernel(
        _sc_body,
        mesh=mesh,
        out_type=jax.ShapeDtypeStruct((D, NJ, NI), jnp.float32),
        scratch_types=[
            pltpu.VMEM((2, JB, IB), jnp.int32),
            pltpu.VMEM((D, VP), jnp.float32),
            pltpu.VMEM((2, D, JB, IB), jnp.float32),
            pltpu.SemaphoreType.DMA,
            pltpu.SemaphoreType.DMA,
        ],
        compiler_params=pltpu.CompilerParams(needs_layout_passes=False),
    )
    return f(x_t, tbl_t)


def kernel(x, table):
    tbl_t = jnp.zeros((D, VP), table.dtype).at[:, :VOC].set(table.T)
    out_t = _sc_lookup(x.T, tbl_t)
    return out_t.transpose(2, 1, 0)
